# baseline (speedup 1.0000x reference)
import functools

import numpy as np

import jax
import jax.numpy as jnp
from jax.experimental import pallas as pl
from jax.experimental.pallas import tpu as pltpu


_SENT_TILE = 8
_PERM_TILE = 256


def _round_up(x, m):
    return ((x + m - 1) // m) * m


def _sum_project_kernel(feat_ref, w_ref, q_ref, *, inv_tokens):
    sums = jnp.sum(feat_ref[...], axis=1) * inv_tokens
    q_ref[...] = jnp.dot(sums, w_ref[...], preferred_element_type=jnp.float32)


def _pair_tanh_kernel(memb_ref, q_ref, b_ref, out_ref):
    pooled = jnp.dot(memb_ref[...], q_ref[...],
                     preferred_element_type=jnp.float32)
    out_ref[...] = jnp.tanh(pooled + b_ref[...])


def _pair_membership(num_sentences, sk, p_pad):
    pairs = [(a, c) for a in range(num_sentences)
             for c in range(a + 1, num_sentences)]
    memb = np.zeros((p_pad, sk), np.float32)
    for i, (a, c) in enumerate(pairs):
        memb[i, a] = 1.0
        memb[i, c] = 1.0
    return len(pairs), memb


def kernel(features, w, b):
    s, nw, d = features.shape
    reasoning_steps = 2
    inv_tokens = 1.0 / float(reasoning_steps * nw)

    s8 = _round_up(s, _SENT_TILE)
    feats = features.astype(jnp.float32)
    if s8 != s:
        feats = jnp.pad(feats, ((0, s8 - s), (0, 0), (0, 0)))

    q = pl.pallas_call(
        functools.partial(_sum_project_kernel, inv_tokens=inv_tokens),
        out_shape=jax.ShapeDtypeStruct((s8, d), jnp.float32),
        grid=(s8 // _SENT_TILE,),
        in_specs=[
            pl.BlockSpec((_SENT_TILE, nw, d), lambda i: (i, 0, 0)),
            pl.BlockSpec((d, d), lambda i: (0, 0)),
        ],
        out_specs=pl.BlockSpec((_SENT_TILE, d), lambda i: (i, 0)),
        compiler_params=pltpu.CompilerParams(
            dimension_semantics=("parallel",)),
    )(feats, w)

    sk = _round_up(s8, 128)
    if sk != s8:
        q = jnp.pad(q, ((0, sk - s8), (0, 0)))

    p_pad = _round_up(s * (s - 1) // 2, _PERM_TILE)
    p, memb_np = _pair_membership(s, sk, p_pad)
    memb = jnp.asarray(memb_np)

    out = pl.pallas_call(
        _pair_tanh_kernel,
        out_shape=jax.ShapeDtypeStruct((p_pad, d), jnp.float32),
        grid=(p_pad // _PERM_TILE,),
        in_specs=[
            pl.BlockSpec((_PERM_TILE, sk), lambda i: (i, 0)),
            pl.BlockSpec((sk, d), lambda i: (0, 0)),
            pl.BlockSpec((1, d), lambda i: (0, 0)),
        ],
        out_specs=pl.BlockSpec((_PERM_TILE, d), lambda i: (i, 0)),
        compiler_params=pltpu.CompilerParams(
            dimension_semantics=("parallel",)),
    )(memb, q, b)
    return out[:p]


# speedup vs baseline: 1.4443x; 1.4443x over previous
import functools

import numpy as np

import jax
import jax.numpy as jnp
from jax.experimental import pallas as pl
from jax.experimental.pallas import tpu as pltpu


_SENT_TILE = 8


def _round_up(x, m):
    return ((x + m - 1) // m) * m


def _pick_perm_tile(p):
    for t in range(min(p, 512), 7, -1):
        if p % t == 0 and t % 8 == 0:
            return t
    return None


def _sum_project_kernel(feat_ref, w_ref, q_ref, *, inv_tokens):
    sums = jnp.sum(feat_ref[...], axis=1) * inv_tokens
    q_ref[...] = jnp.dot(sums, w_ref[...], preferred_element_type=jnp.float32)


def _pair_tanh_kernel(memb_ref, q_ref, b_ref, out_ref):
    pooled = jnp.dot(memb_ref[...], q_ref[...],
                     preferred_element_type=jnp.float32)
    out_ref[...] = jnp.tanh(pooled + b_ref[...])


def _pair_membership(num_sentences, sk, p_pad):
    pairs = [(a, c) for a in range(num_sentences)
             for c in range(a + 1, num_sentences)]
    memb = np.zeros((p_pad, sk), np.float32)
    for i, (a, c) in enumerate(pairs):
        memb[i, a] = 1.0
        memb[i, c] = 1.0
    return len(pairs), memb


def kernel(features, w, b):
    s, nw, d = features.shape
    reasoning_steps = 2
    inv_tokens = 1.0 / float(reasoning_steps * nw)

    s8 = _round_up(s, _SENT_TILE)
    feats = features.astype(jnp.float32)
    if s8 != s:
        feats = jnp.pad(feats, ((0, s8 - s), (0, 0), (0, 0)))

    q = pl.pallas_call(
        functools.partial(_sum_project_kernel, inv_tokens=inv_tokens),
        out_shape=jax.ShapeDtypeStruct((s8, d), jnp.float32),
        grid=(s8 // _SENT_TILE,),
        in_specs=[
            pl.BlockSpec((_SENT_TILE, nw, d), lambda i: (i, 0, 0)),
            pl.BlockSpec((d, d), lambda i: (0, 0)),
        ],
        out_specs=pl.BlockSpec((_SENT_TILE, d), lambda i: (i, 0)),
        compiler_params=pltpu.CompilerParams(
            dimension_semantics=("parallel",)),
    )(feats, w)

    sk = s8
    p = s * (s - 1) // 2
    tile = _pick_perm_tile(p)
    if tile is None:
        tile = 256
        p_pad = _round_up(p, tile)
    else:
        p_pad = p
    _, memb_np = _pair_membership(s, sk, p_pad)
    memb = jnp.asarray(memb_np)

    out = pl.pallas_call(
        _pair_tanh_kernel,
        out_shape=jax.ShapeDtypeStruct((p_pad, d), jnp.float32),
        grid=(p_pad // tile,),
        in_specs=[
            pl.BlockSpec((tile, sk), lambda i: (i, 0)),
            pl.BlockSpec((sk, d), lambda i: (0, 0)),
            pl.BlockSpec((1, d), lambda i: (0, 0)),
        ],
        out_specs=pl.BlockSpec((tile, d), lambda i: (i, 0)),
        compiler_params=pltpu.CompilerParams(
            dimension_semantics=("parallel",)),
    )(memb, q, b)
    return out if p_pad == p else out[:p]


# speedup vs baseline: 1.5296x; 1.0590x over previous
import functools

import numpy as np

import jax
import jax.numpy as jnp
from jax.experimental import pallas as pl
from jax.experimental.pallas import tpu as pltpu


_SENT_TILE = 16


def _round_up(x, m):
    return ((x + m - 1) // m) * m


def _pick_perm_tile(p):
    for t in range(min(p, 512), 7, -1):
        if p % t == 0 and t % 8 == 0:
            return t
    return None


def _fused_kernel(feat_ref, w_ref, memb_ref, b_ref, out_ref, q_ref, *,
                  inv_tokens, nsum):
    i = pl.program_id(0)
    ts = feat_ref.shape[0]

    @pl.when(i < nsum)
    def _sum_project():
        sums = jnp.sum(feat_ref[...], axis=1) * inv_tokens
        q_ref[pl.ds(i * ts, ts), :] = jnp.dot(
            sums, w_ref[...], preferred_element_type=jnp.float32)

    @pl.when(i >= nsum)
    def _pair_tanh():
        pooled = jnp.dot(memb_ref[...], q_ref[...],
                         preferred_element_type=jnp.float32)
        out_ref[...] = jnp.tanh(pooled + b_ref[...])


def _pair_membership(num_sentences, sk, p_pad):
    pairs = [(a, c) for a in range(num_sentences)
             for c in range(a + 1, num_sentences)]
    memb = np.zeros((p_pad, sk), np.float32)
    for i, (a, c) in enumerate(pairs):
        memb[i, a] = 1.0
        memb[i, c] = 1.0
    return len(pairs), memb


def kernel(features, w, b):
    s, nw, d = features.shape
    reasoning_steps = 2
    inv_tokens = 1.0 / float(reasoning_steps * nw)

    s8 = _round_up(s, _SENT_TILE)
    feats = features.astype(jnp.float32)
    if s8 != s:
        feats = jnp.pad(feats, ((0, s8 - s), (0, 0), (0, 0)))
    nsum = s8 // _SENT_TILE

    p = s * (s - 1) // 2
    tile = _pick_perm_tile(p)
    if tile is None:
        tile = 256
        p_pad = _round_up(p, tile)
    else:
        p_pad = p
    npair = p_pad // tile
    _, memb_np = _pair_membership(s, s8, p_pad)
    memb = jnp.asarray(memb_np)

    fused = functools.partial(_fused_kernel, inv_tokens=inv_tokens,
                              nsum=nsum)
    out = pl.pallas_call(
        fused,
        out_shape=jax.ShapeDtypeStruct((p_pad, d), jnp.float32),
        grid=(nsum + npair,),
        in_specs=[
            pl.BlockSpec((_SENT_TILE, nw, d),
                         lambda i: (jnp.minimum(i, nsum - 1), 0, 0)),
            pl.BlockSpec((d, d), lambda i: (0, 0)),
            pl.BlockSpec((tile, s8),
                         lambda i: (jnp.maximum(i - nsum, 0), 0)),
            pl.BlockSpec((1, d), lambda i: (0, 0)),
        ],
        out_specs=pl.BlockSpec((tile, d),
                               lambda i: (jnp.maximum(i - nsum, 0), 0)),
        scratch_shapes=[pltpu.VMEM((s8, d), jnp.float32)],
        compiler_params=pltpu.CompilerParams(
            dimension_semantics=("arbitrary",)),
    )(feats, w, memb, b)
    return out if p_pad == p else out[:p]


# speedup vs baseline: 1.6175x; 1.0575x over previous
import functools

import numpy as np

import jax
import jax.numpy as jnp
from jax.experimental import pallas as pl
from jax.experimental.pallas import tpu as pltpu


_SENT_TILE = 16


def _round_up(x, m):
    return ((x + m - 1) // m) * m


def _pick_perm_tile(p):
    for t in range(min(p, 1008), 7, -1):
        if p % t == 0 and t % 8 == 0:
            return t
    return None


def _fused_kernel(feat_ref, w_ref, memb_ref, b_ref, out_ref, q_ref, *,
                  inv_tokens, nsum):
    i = pl.program_id(0)
    ts = feat_ref.shape[0]

    @pl.when(i < nsum)
    def _sum_project():
        sums = jnp.sum(feat_ref[...], axis=1) * inv_tokens
        q_ref[pl.ds(i * ts, ts), :] = jnp.dot(
            sums, w_ref[...], preferred_element_type=jnp.float32)

    @pl.when(i >= nsum)
    def _pair_tanh():
        pooled = jnp.dot(memb_ref[...], q_ref[...],
                         preferred_element_type=jnp.float32)
        out_ref[...] = jnp.tanh(pooled + b_ref[...])


def _pair_membership(num_sentences, sk, p_pad):
    pairs = [(a, c) for a in range(num_sentences)
             for c in range(a + 1, num_sentences)]
    memb = np.zeros((p_pad, sk), np.float32)
    for i, (a, c) in enumerate(pairs):
        memb[i, a] = 1.0
        memb[i, c] = 1.0
    return len(pairs), memb


def kernel(features, w, b):
    s, nw, d = features.shape
    reasoning_steps = 2
    inv_tokens = 1.0 / float(reasoning_steps * nw)

    s8 = _round_up(s, _SENT_TILE)
    feats = features.astype(jnp.float32)
    if s8 != s:
        feats = jnp.pad(feats, ((0, s8 - s), (0, 0), (0, 0)))
    nsum = s8 // _SENT_TILE

    p = s * (s - 1) // 2
    tile = _pick_perm_tile(p)
    if tile is None:
        tile = 256
        p_pad = _round_up(p, tile)
    else:
        p_pad = p
    npair = p_pad // tile
    _, memb_np = _pair_membership(s, s8, p_pad)
    memb = jnp.asarray(memb_np)

    fused = functools.partial(_fused_kernel, inv_tokens=inv_tokens,
                              nsum=nsum)
    out = pl.pallas_call(
        fused,
        out_shape=jax.ShapeDtypeStruct((p_pad, d), jnp.float32),
        grid=(nsum + npair,),
        in_specs=[
            pl.BlockSpec((_SENT_TILE, nw, d),
                         lambda i: (jnp.minimum(i, nsum - 1), 0, 0)),
            pl.BlockSpec((d, d), lambda i: (0, 0)),
            pl.BlockSpec((tile, s8),
                         lambda i: (jnp.maximum(i - nsum, 0), 0)),
            pl.BlockSpec((1, d), lambda i: (0, 0)),
        ],
        out_specs=pl.BlockSpec((tile, d),
                               lambda i: (jnp.maximum(i - nsum, 0), 0)),
        scratch_shapes=[pltpu.VMEM((s8, d), jnp.float32)],
        compiler_params=pltpu.CompilerParams(
            dimension_semantics=("arbitrary",)),
    )(feats, w, memb, b)
    return out if p_pad == p else out[:p]


# speedup vs baseline: 1.6345x; 1.0105x over previous
import functools

import numpy as np

import jax
import jax.numpy as jnp
from jax.experimental import pallas as pl
from jax.experimental.pallas import tpu as pltpu


_SENT_TILE = 16


def _round_up(x, m):
    return ((x + m - 1) // m) * m


def _pick_perm_tile(p):
    for t in range(min(p, 1008), 7, -1):
        if p % t == 0 and t % 8 == 0:
            return t
    return None


def _fused_kernel(feat_ref, w_ref, memb_ref, b_ref, out_ref, q_ref, *,
                  inv_tokens, nsum):
    i = pl.program_id(0)
    ts = feat_ref.shape[0]

    @pl.when(i < nsum)
    def _sum_project():
        sums = jnp.sum(feat_ref[...], axis=1) * inv_tokens
        q = jnp.dot(sums, w_ref[...], preferred_element_type=jnp.float32)
        q_ref[pl.ds(i * ts, ts), :] = q.astype(jnp.bfloat16)

    @pl.when(i >= nsum)
    def _pair_tanh():
        pooled = jnp.dot(memb_ref[...], q_ref[...],
                         preferred_element_type=jnp.float32)
        out_ref[...] = jnp.tanh(pooled + b_ref[...])


def _pair_membership(num_sentences, sk, p_pad):
    pairs = [(a, c) for a in range(num_sentences)
             for c in range(a + 1, num_sentences)]
    memb = np.zeros((p_pad, sk), np.float32)
    for i, (a, c) in enumerate(pairs):
        memb[i, a] = 1.0
        memb[i, c] = 1.0
    return len(pairs), memb


def kernel(features, w, b):
    s, nw, d = features.shape
    reasoning_steps = 2
    inv_tokens = 1.0 / float(reasoning_steps * nw)

    s8 = _round_up(s, _SENT_TILE)
    feats = features.astype(jnp.float32)
    if s8 != s:
        feats = jnp.pad(feats, ((0, s8 - s), (0, 0), (0, 0)))
    nsum = s8 // _SENT_TILE

    p = s * (s - 1) // 2
    tile = _pick_perm_tile(p)
    if tile is None:
        tile = 256
        p_pad = _round_up(p, tile)
    else:
        p_pad = p
    npair = p_pad // tile
    _, memb_np = _pair_membership(s, s8, p_pad)
    memb = jnp.asarray(memb_np).astype(jnp.bfloat16)

    fused = functools.partial(_fused_kernel, inv_tokens=inv_tokens,
                              nsum=nsum)
    out = pl.pallas_call(
        fused,
        out_shape=jax.ShapeDtypeStruct((p_pad, d), jnp.float32),
        grid=(nsum + npair,),
        in_specs=[
            pl.BlockSpec((_SENT_TILE, nw, d),
                         lambda i: (jnp.minimum(i, nsum - 1), 0, 0)),
            pl.BlockSpec((d, d), lambda i: (0, 0)),
            pl.BlockSpec((tile, s8),
                         lambda i: (jnp.maximum(i - nsum, 0), 0)),
            pl.BlockSpec((1, d), lambda i: (0, 0)),
        ],
        out_specs=pl.BlockSpec((tile, d),
                               lambda i: (jnp.maximum(i - nsum, 0), 0)),
        scratch_shapes=[pltpu.VMEM((s8, d), jnp.bfloat16)],
        compiler_params=pltpu.CompilerParams(
            dimension_semantics=("arbitrary",)),
    )(feats, w, memb, b)
    return out if p_pad == p else out[:p]


# speedup vs baseline: 1.7364x; 1.0624x over previous
import functools

import numpy as np

import jax
import jax.numpy as jnp
from jax.experimental import pallas as pl
from jax.experimental.pallas import tpu as pltpu


_SENT_TILE = 16
_FEAT_BUFS = 3
_OUT_BUFS = 2


def _round_up(x, m):
    return ((x + m - 1) // m) * m


def _pick_perm_tile(p):
    for t in range(min(p, 1008), 7, -1):
        if p % t == 0 and t % 8 == 0:
            return t
    return None


def _fused_kernel(feat_hbm, w_ref, memb_ref, b_ref, out_hbm,
                  fbuf, obuf, q_ref, fsem, osem, *,
                  inv_tokens, ts, nsum, tile, npair):
    for k in range(min(_FEAT_BUFS, nsum)):
        pltpu.make_async_copy(feat_hbm.at[pl.ds(k * ts, ts)],
                              fbuf.at[k], fsem.at[k]).start()
    for k in range(nsum):
        slot = k % _FEAT_BUFS
        pltpu.make_async_copy(fbuf.at[slot], fbuf.at[slot],
                              fsem.at[slot]).wait()
        sums = jnp.sum(fbuf[slot], axis=1) * inv_tokens
        q = jnp.dot(sums, w_ref[...], preferred_element_type=jnp.float32)
        q_ref[pl.ds(k * ts, ts), :] = q.astype(jnp.bfloat16)
        nxt = k + _FEAT_BUFS
        if nxt < nsum:
            pltpu.make_async_copy(feat_hbm.at[pl.ds(nxt * ts, ts)],
                                  fbuf.at[slot], fsem.at[slot]).start()

    for j in range(npair):
        oslot = j % _OUT_BUFS
        if j >= _OUT_BUFS:
            pltpu.make_async_copy(obuf.at[oslot], obuf.at[oslot],
                                  osem.at[oslot]).wait()
        pooled = jnp.dot(memb_ref[j * tile:(j + 1) * tile, :], q_ref[...],
                         preferred_element_type=jnp.float32)
        obuf[oslot] = jnp.tanh(pooled + b_ref[...])
        pltpu.make_async_copy(obuf.at[oslot],
                              out_hbm.at[pl.ds(j * tile, tile)],
                              osem.at[oslot]).start()
    for j in range(max(npair - _OUT_BUFS, 0), npair):
        oslot = j % _OUT_BUFS
        pltpu.make_async_copy(obuf.at[oslot], obuf.at[oslot],
                              osem.at[oslot]).wait()


def _pair_membership(num_sentences, sk, p_pad):
    pairs = [(a, c) for a in range(num_sentences)
             for c in range(a + 1, num_sentences)]
    memb = np.zeros((p_pad, sk), np.float32)
    for i, (a, c) in enumerate(pairs):
        memb[i, a] = 1.0
        memb[i, c] = 1.0
    return len(pairs), memb


def kernel(features, w, b):
    s, nw, d = features.shape
    reasoning_steps = 2
    inv_tokens = 1.0 / float(reasoning_steps * nw)

    s8 = _round_up(s, _SENT_TILE)
    feats = features.astype(jnp.float32)
    if s8 != s:
        feats = jnp.pad(feats, ((0, s8 - s), (0, 0), (0, 0)))
    nsum = s8 // _SENT_TILE

    p = s * (s - 1) // 2
    tile = _pick_perm_tile(p)
    if tile is None:
        tile = 256
        p_pad = _round_up(p, tile)
    else:
        p_pad = p
    npair = p_pad // tile
    _, memb_np = _pair_membership(s, s8, p_pad)
    memb = jnp.asarray(memb_np).astype(jnp.bfloat16)

    fused = functools.partial(_fused_kernel, inv_tokens=inv_tokens,
                              ts=_SENT_TILE, nsum=nsum, tile=tile,
                              npair=npair)
    out = pl.pallas_call(
        fused,
        out_shape=jax.ShapeDtypeStruct((p_pad, d), jnp.float32),
        in_specs=[
            pl.BlockSpec(memory_space=pl.ANY),
            pl.BlockSpec(memory_space=pltpu.VMEM),
            pl.BlockSpec(memory_space=pltpu.VMEM),
            pl.BlockSpec(memory_space=pltpu.VMEM),
        ],
        out_specs=pl.BlockSpec(memory_space=pl.ANY),
        scratch_shapes=[
            pltpu.VMEM((_FEAT_BUFS, _SENT_TILE, nw, d), jnp.float32),
            pltpu.VMEM((_OUT_BUFS, tile, d), jnp.float32),
            pltpu.VMEM((s8, d), jnp.bfloat16),
            pltpu.SemaphoreType.DMA((_FEAT_BUFS,)),
            pltpu.SemaphoreType.DMA((_OUT_BUFS,)),
        ],
    )(feats, w, memb, b)
    return out if p_pad == p else out[:p]


# speedup vs baseline: 1.7395x; 1.0018x over previous
import functools

import numpy as np

import jax
import jax.numpy as jnp
from jax.experimental import pallas as pl
from jax.experimental.pallas import tpu as pltpu


_SENT_TILE = 16
_FEAT_BUFS = 4
_OUT_BUFS = 2


def _round_up(x, m):
    return ((x + m - 1) // m) * m


def _pick_perm_tile(p):
    for t in range(min(p, 1008), 7, -1):
        if p % t == 0 and t % 8 == 0:
            return t
    return None


def _sum_project(block, w_ref, inv_tokens):
    sums = jnp.sum(block, axis=1) * inv_tokens
    q = jnp.dot(sums, w_ref[...], preferred_element_type=jnp.float32)
    return q.astype(jnp.bfloat16)


def _fused_kernel(feat_hbm, w_ref, memb1_ref, memb2_ref, b_ref, out_hbm,
                  fbuf, obuf, q_ref, fsem, csem, osem, *,
                  inv_tokens, ts, nsum, tile, npair):
    nfull = nsum - 1
    half = ts // 2
    last_slot = nfull % _FEAT_BUFS

    for k in range(min(_FEAT_BUFS, nfull)):
        pltpu.make_async_copy(feat_hbm.at[pl.ds(k * ts, ts)],
                              fbuf.at[k], fsem.at[k]).start()
    for c in range(2):
        pltpu.make_async_copy(
            feat_hbm.at[pl.ds(nfull * ts + c * half, half)],
            fbuf.at[last_slot, pl.ds(c * half, half)], csem.at[c]).start()

    for k in range(nfull):
        slot = k % _FEAT_BUFS
        pltpu.make_async_copy(fbuf.at[slot], fbuf.at[slot],
                              fsem.at[slot]).wait()
        q_ref[pl.ds(k * ts, ts), :] = _sum_project(fbuf[slot], w_ref,
                                                   inv_tokens)
        nxt = k + _FEAT_BUFS
        if nxt < nfull:
            pltpu.make_async_copy(feat_hbm.at[pl.ds(nxt * ts, ts)],
                                  fbuf.at[slot], fsem.at[slot]).start()

    kpart = nfull * ts
    overlap = npair <= _OUT_BUFS and kpart > 0
    if overlap:
        for j in range(npair):
            obuf[j % _OUT_BUFS] = jnp.dot(
                memb1_ref[j * tile:(j + 1) * tile, :],
                q_ref[:kpart, :], preferred_element_type=jnp.float32)

    for c in range(2):
        pltpu.make_async_copy(fbuf.at[last_slot, pl.ds(c * half, half)],
                              fbuf.at[last_slot, pl.ds(c * half, half)],
                              csem.at[c]).wait()
        block = fbuf[last_slot, pl.ds(c * half, half)]
        q_ref[pl.ds(kpart + c * half, half), :] = _sum_project(
            block, w_ref, inv_tokens)

    for j in range(npair):
        oslot = j % _OUT_BUFS
        if not overlap and j >= _OUT_BUFS:
            pltpu.make_async_copy(obuf.at[oslot], obuf.at[oslot],
                                  osem.at[oslot]).wait()
        if overlap:
            pooled = obuf[oslot] + jnp.dot(
                memb2_ref[j * tile:(j + 1) * tile, :],
                q_ref[kpart:, :], preferred_element_type=jnp.float32)
        elif kpart > 0:
            pooled = (jnp.dot(memb1_ref[j * tile:(j + 1) * tile, :],
                              q_ref[:kpart, :],
                              preferred_element_type=jnp.float32)
                      + jnp.dot(memb2_ref[j * tile:(j + 1) * tile, :],
                                q_ref[kpart:, :],
                                preferred_element_type=jnp.float32))
        else:
            pooled = jnp.dot(memb2_ref[j * tile:(j + 1) * tile, :],
                             q_ref[...], preferred_element_type=jnp.float32)
        obuf[oslot] = jnp.tanh(pooled + b_ref[...])
        pltpu.make_async_copy(obuf.at[oslot],
                              out_hbm.at[pl.ds(j * tile, tile)],
                              osem.at[oslot]).start()
    for j in range(max(npair - _OUT_BUFS, 0), npair):
        oslot = j % _OUT_BUFS
        pltpu.make_async_copy(obuf.at[oslot], obuf.at[oslot],
                              osem.at[oslot]).wait()


def _pair_membership(num_sentences, sk, p_pad):
    pairs = [(a, c) for a in range(num_sentences)
             for c in range(a + 1, num_sentences)]
    memb = np.zeros((p_pad, sk), np.float32)
    for i, (a, c) in enumerate(pairs):
        memb[i, a] = 1.0
        memb[i, c] = 1.0
    return len(pairs), memb


def kernel(features, w, b):
    s, nw, d = features.shape
    reasoning_steps = 2
    inv_tokens = 1.0 / float(reasoning_steps * nw)

    s8 = _round_up(s, _SENT_TILE)
    feats = features.astype(jnp.float32)
    if s8 != s:
        feats = jnp.pad(feats, ((0, s8 - s), (0, 0), (0, 0)))
    nsum = s8 // _SENT_TILE

    p = s * (s - 1) // 2
    tile = _pick_perm_tile(p)
    if tile is None:
        tile = 256
        p_pad = _round_up(p, tile)
    else:
        p_pad = p
    npair = p_pad // tile
    _, memb_np = _pair_membership(s, s8, p_pad)
    kpart = (nsum - 1) * _SENT_TILE
    memb1 = jnp.asarray(memb_np[:, :max(kpart, 1)]).astype(jnp.bfloat16)
    memb2 = jnp.asarray(memb_np[:, kpart:]).astype(jnp.bfloat16)

    fused = functools.partial(_fused_kernel, inv_tokens=inv_tokens,
                              ts=_SENT_TILE, nsum=nsum, tile=tile,
                              npair=npair)
    out = pl.pallas_call(
        fused,
        out_shape=jax.ShapeDtypeStruct((p_pad, d), jnp.float32),
        in_specs=[
            pl.BlockSpec(memory_space=pl.ANY),
            pl.BlockSpec(memory_space=pltpu.VMEM),
            pl.BlockSpec(memory_space=pltpu.VMEM),
            pl.BlockSpec(memory_space=pltpu.VMEM),
            pl.BlockSpec(memory_space=pltpu.VMEM),
        ],
        out_specs=pl.BlockSpec(memory_space=pl.ANY),
        scratch_shapes=[
            pltpu.VMEM((_FEAT_BUFS, _SENT_TILE, nw, d), jnp.float32),
            pltpu.VMEM((_OUT_BUFS, tile, d), jnp.float32),
            pltpu.VMEM((s8, d), jnp.bfloat16),
            pltpu.SemaphoreType.DMA((_FEAT_BUFS,)),
            pltpu.SemaphoreType.DMA((2,)),
            pltpu.SemaphoreType.DMA((_OUT_BUFS,)),
        ],
    )(feats, w, memb1, memb2, b)
    return out if p_pad == p else out[:p]
